# baseline (speedup 1.0000x reference)
import functools

import jax
import jax.numpy as jnp
from jax import lax
from jax.experimental import pallas as pl
from jax.experimental.pallas import tpu as pltpu
from jax.experimental.pallas import tpu_sc as plsc

N_NODES = 10000
N_EDGES = 320000
NODE_DIM = 128
EDGE_DIM = 16
OUT_DIM = 128
Z_DIM = NODE_DIM + EDGE_DIM
HALF = NODE_DIM // 2

NC = 2
NS = 16
NW = NC * NS

CHUNK = 128
NCHUNKS = N_EDGES // CHUNK
K = 158
KLAST = NCHUNKS - (NS - 1) * K
HK = K // 2
N_PAD = 10112
DW = 16
ROWS_PER_TILE = N_PAD // NS


def _sc_scatter_kernel(x2_hbm, eidx_hbm, e_hbm, px_hbm, pe_hbm, pd_hbm,
                       src_v, dst_v, r0, r1, e_v, ones_v,
                       acc_x, acc_e, acc_d,
                       gsem0, gsem1, edsem):
  rows0, rows1 = r0, r1
  cid = lax.axis_index("c")
  sid = lax.axis_index("s")

  zero = jnp.zeros((16,), jnp.float32)
  one = jnp.ones((16,), jnp.float32)

  def init_buf(i, carry):
    for l in range(HALF // 16):
      r0[i, pl.ds(l * 16, 16)] = zero
    e_v[i, pl.ds(0, 16)] = zero
    ones_v[i, pl.ds(0, 16)] = one
    return carry

  lax.fori_loop(0, CHUNK, init_buf, 0)

  row0 = sid * ROWS_PER_TILE
  full, rem = divmod(ROWS_PER_TILE, CHUNK)
  for t in range(full):
    pltpu.sync_copy(rows0, acc_x.at[pl.ds(row0 + t * CHUNK, CHUNK)])
    pltpu.sync_copy(e_v, acc_e.at[pl.ds(row0 + t * CHUNK, CHUNK)])
    pltpu.sync_copy(e_v, acc_d.at[pl.ds(row0 + t * CHUNK, CHUNK)])
  if rem:
    pltpu.sync_copy(rows0.at[pl.ds(0, rem)],
                    acc_x.at[pl.ds(row0 + full * CHUNK, rem)])
    pltpu.sync_copy(e_v.at[pl.ds(0, rem)],
                    acc_e.at[pl.ds(row0 + full * CHUNK, rem)])
    pltpu.sync_copy(e_v.at[pl.ds(0, rem)],
                    acc_d.at[pl.ds(row0 + full * CHUNK, rem)])
  plsc.subcore_barrier()

  base = sid * K
  nv = jnp.where(sid == NS - 1, KLAST, K)

  @pl.when(sid < NS - 1)
  def _():
    pltpu.sync_copy(eidx_hbm.at[0, pl.ds(base, K)], src_v)
    pltpu.sync_copy(eidx_hbm.at[1, pl.ds(base, K)], dst_v)

  @pl.when(sid == NS - 1)
  def _():
    pltpu.sync_copy(eidx_hbm.at[0, pl.ds(base, KLAST)],
                    src_v.at[pl.ds(0, KLAST)])
    pltpu.sync_copy(eidx_hbm.at[1, pl.ds(base, KLAST)],
                    dst_v.at[pl.ds(0, KLAST)])

  def to_half_row(t, carry):
    i = t // (CHUNK // 16)
    l = t % (CHUNK // 16)
    v = src_v[i, pl.ds(l * 16, 16)]
    src_v[i, pl.ds(l * 16, 16)] = v + v + cid
    return carry

  lax.fori_loop(0, nv * (CHUNK // 16), to_half_row, 0)

  on_core0 = cid == 0

  def emine_guarded(j, pend):
    mine = jnp.where(on_core0, j < HK, j >= HK)

    @pl.when(mine)
    def _():
      @pl.when(pend > 0)
      def _():
        pltpu.make_async_copy(e_v, acc_e.at[dst_v.at[j]], edsem).wait()
        pltpu.make_async_copy(ones_v, acc_d.at[dst_v.at[j]], edsem).wait()

      pltpu.sync_copy(e_hbm.at[pl.ds((base + j) * CHUNK, CHUNK)], e_v)
      pltpu.async_copy(e_v, acc_e.at[dst_v.at[j]], edsem, add=True)
      pltpu.async_copy(ones_v, acc_d.at[dst_v.at[j]], edsem, add=True)

    return jnp.where(mine, jnp.int32(1), pend)

  def process(j, buf, pend):
    pltpu.sync_copy(buf, acc_x.at[dst_v.at[j]], add=True)
    return emine_guarded(j, pend)

  pltpu.async_copy(x2_hbm.at[src_v.at[0]], rows0, gsem0)

  def body(i, pend):
    g = i * 2
    pltpu.async_copy(x2_hbm.at[src_v.at[g + 1]], rows1, gsem1)
    pltpu.make_async_copy(x2_hbm.at[src_v.at[g]], rows0, gsem0).wait()
    pend = process(g, rows0, pend)

    @pl.when(g + 2 < nv)
    def _():
      pltpu.async_copy(x2_hbm.at[src_v.at[g + 2]], rows0, gsem0)

    pltpu.make_async_copy(x2_hbm.at[src_v.at[g + 1]], rows1, gsem1).wait()
    pend = process(g + 1, rows1, pend)
    return pend

  pend = lax.fori_loop(0, nv // 2, body, jnp.int32(0))

  @pl.when(pend > 0)
  def _():
    pltpu.make_async_copy(e_v, acc_e.at[dst_v.at[0]], edsem).wait()
    pltpu.make_async_copy(ones_v, acc_d.at[dst_v.at[0]], edsem).wait()

  plsc.subcore_barrier()

  pltpu.sync_copy(acc_x.at[pl.ds(row0, ROWS_PER_TILE)],
                  px_hbm.at[cid, pl.ds(row0, ROWS_PER_TILE)])
  pltpu.sync_copy(acc_e.at[pl.ds(row0, ROWS_PER_TILE)],
                  pe_hbm.at[cid, pl.ds(row0, ROWS_PER_TILE)])
  pltpu.sync_copy(acc_d.at[pl.ds(row0, ROWS_PER_TILE)],
                  pd_hbm.at[cid, pl.ds(row0, ROWS_PER_TILE)])


@functools.cache
def _sc_scatter():
  return pl.kernel(
      _sc_scatter_kernel,
      out_type=[
          jax.ShapeDtypeStruct((NC, N_PAD, HALF), jnp.float32),
          jax.ShapeDtypeStruct((NC, N_PAD, EDGE_DIM), jnp.float32),
          jax.ShapeDtypeStruct((NC, N_PAD, DW), jnp.float32),
      ],
      mesh=plsc.VectorSubcoreMesh(
          core_axis_name="c", subcore_axis_name="s",
          num_cores=NC, num_subcores=NS),
      scratch_types=[
          pltpu.VMEM((K, CHUNK), jnp.int32),
          pltpu.VMEM((K, CHUNK), jnp.int32),
          pltpu.VMEM((CHUNK, HALF), jnp.float32),
          pltpu.VMEM((CHUNK, HALF), jnp.float32),
          pltpu.VMEM((CHUNK, EDGE_DIM), jnp.float32),
          pltpu.VMEM((CHUNK, DW), jnp.float32),
          pltpu.VMEM_SHARED((N_PAD, HALF), jnp.float32),
          pltpu.VMEM_SHARED((N_PAD, EDGE_DIM), jnp.float32),
          pltpu.VMEM_SHARED((N_PAD, DW), jnp.float32),
      ] + [pltpu.SemaphoreType.DMA] * 3,
      compiler_params=pltpu.CompilerParams(use_tc_tiling_on_sc=False),
  )


def _dense_body(px_ref, pe_ref, pd_ref, x_ref, z0_ref, wpre_ref, bpre_ref,
                wupd_ref, bupd_ref, out_ref):
  sx = jnp.concatenate([px_ref[0], px_ref[1]], axis=1)
  se = pe_ref[0] + pe_ref[1]
  deg = (pd_ref[0] + pd_ref[1])[:, :1]
  agg = (
      jnp.dot(sx, wpre_ref[:NODE_DIM, :], preferred_element_type=jnp.float32)
      + jnp.dot(se, wpre_ref[NODE_DIM:, :],
                preferred_element_type=jnp.float32)
      + deg * bpre_ref[0, :][None, :]
  )
  z = jnp.where(deg > 0, jnp.maximum(agg, 0.0), z0_ref[...])
  h = (
      jnp.dot(x_ref[...], wupd_ref[:NODE_DIM, :],
              preferred_element_type=jnp.float32)
      + jnp.dot(z, wupd_ref[NODE_DIM:, :], preferred_element_type=jnp.float32)
      + bupd_ref[0, :][None, :]
  )
  out_ref[...] = jnp.maximum(h, 0.0)


BLK = 1024

_dense = pl.pallas_call(
    _dense_body,
    grid=(pl.cdiv(N_PAD, BLK),),
    in_specs=[
        pl.BlockSpec((NC, BLK, HALF), lambda i: (0, i, 0)),
        pl.BlockSpec((NC, BLK, EDGE_DIM), lambda i: (0, i, 0)),
        pl.BlockSpec((NC, BLK, DW), lambda i: (0, i, 0)),
        pl.BlockSpec((BLK, NODE_DIM), lambda i: (i, 0)),
        pl.BlockSpec((BLK, Z_DIM), lambda i: (i, 0)),
        pl.BlockSpec((Z_DIM, Z_DIM), lambda i: (0, 0)),
        pl.BlockSpec((1, Z_DIM), lambda i: (0, 0)),
        pl.BlockSpec((NODE_DIM + Z_DIM, OUT_DIM), lambda i: (0, 0)),
        pl.BlockSpec((1, OUT_DIM), lambda i: (0, 0)),
    ],
    out_specs=pl.BlockSpec((BLK, OUT_DIM), lambda i: (i, 0)),
    out_shape=jax.ShapeDtypeStruct((N_NODES, OUT_DIM), jnp.float32),
)


def kernel(x, edge_attr, edge_index, z_init, W_pre, b_pre, W_upd, b_upd):
  eidx = edge_index.astype(jnp.int32).reshape(2, NCHUNKS, CHUNK)
  x2 = x.reshape(NC * N_NODES, HALF)

  px, pe, pd = _sc_scatter()(x2, eidx, edge_attr)

  h = _dense(px, pe, pd, x, z_init, W_pre, b_pre.reshape(1, -1),
             W_upd, b_upd.reshape(1, -1))
  return h


# speedup vs baseline: 1.0519x; 1.0519x over previous
import functools

import jax
import jax.numpy as jnp
from jax import lax
from jax.experimental import pallas as pl
from jax.experimental.pallas import tpu as pltpu
from jax.experimental.pallas import tpu_sc as plsc

N_NODES = 10000
N_EDGES = 320000
NODE_DIM = 128
EDGE_DIM = 16
OUT_DIM = 128
Z_DIM = NODE_DIM + EDGE_DIM
HALF = NODE_DIM // 2

NC = 2
NS = 16
NW = NC * NS

CHUNK = 128
NCHUNKS = N_EDGES // CHUNK
K = 158
KLAST = NCHUNKS - (NS - 1) * K
HK = K // 2
N_PAD = 10112
DW = 16
ROWS_PER_TILE = N_PAD // NS


def _sc_scatter_kernel(x2_hbm, eidx_hbm, e_hbm, px_hbm, ped_hbm,
                       src_v, dst_v, r0, r1, e_v, ones_v,
                       acc_x, acc_e, acc_d,
                       gsem0, gsem1, edsem):
  rows0, rows1 = r0, r1
  cid = lax.axis_index("c")
  sid = lax.axis_index("s")

  zero = jnp.zeros((16,), jnp.float32)
  one = jnp.ones((16,), jnp.float32)

  def init_buf(i, carry):
    for l in range(HALF // 16):
      r0[i, pl.ds(l * 16, 16)] = zero
    e_v[i, pl.ds(0, 16)] = zero
    ones_v[i, pl.ds(0, 16)] = one
    return carry

  lax.fori_loop(0, CHUNK, init_buf, 0)

  row0 = sid * ROWS_PER_TILE
  full, rem = divmod(ROWS_PER_TILE, CHUNK)
  for t in range(full):
    pltpu.sync_copy(rows0, acc_x.at[pl.ds(row0 + t * CHUNK, CHUNK)])
    pltpu.sync_copy(e_v, acc_e.at[pl.ds(row0 + t * CHUNK, CHUNK)])
    pltpu.sync_copy(e_v, acc_d.at[pl.ds(row0 + t * CHUNK, CHUNK)])
  if rem:
    pltpu.sync_copy(rows0.at[pl.ds(0, rem)],
                    acc_x.at[pl.ds(row0 + full * CHUNK, rem)])
    pltpu.sync_copy(e_v.at[pl.ds(0, rem)],
                    acc_e.at[pl.ds(row0 + full * CHUNK, rem)])
    pltpu.sync_copy(e_v.at[pl.ds(0, rem)],
                    acc_d.at[pl.ds(row0 + full * CHUNK, rem)])
  plsc.subcore_barrier()

  base = sid * K
  nv = jnp.where(sid == NS - 1, KLAST, K)

  @pl.when(sid < NS - 1)
  def _():
    pltpu.sync_copy(eidx_hbm.at[0, pl.ds(base, K)], src_v)
    pltpu.sync_copy(eidx_hbm.at[1, pl.ds(base, K)], dst_v)

  @pl.when(sid == NS - 1)
  def _():
    pltpu.sync_copy(eidx_hbm.at[0, pl.ds(base, KLAST)],
                    src_v.at[pl.ds(0, KLAST)])
    pltpu.sync_copy(eidx_hbm.at[1, pl.ds(base, KLAST)],
                    dst_v.at[pl.ds(0, KLAST)])

  def to_half_row(t, carry):
    i = t // (CHUNK // 16)
    l = t % (CHUNK // 16)
    v = src_v[i, pl.ds(l * 16, 16)]
    src_v[i, pl.ds(l * 16, 16)] = v + v + cid
    return carry

  lax.fori_loop(0, nv * (CHUNK // 16), to_half_row, 0)

  on_core0 = cid == 0

  def emine_guarded(j, pend):
    mine = jnp.where(on_core0, j < HK, j >= HK)

    @pl.when(mine)
    def _():
      @pl.when(pend > 0)
      def _():
        pltpu.make_async_copy(e_v, acc_e.at[dst_v.at[j]], edsem).wait()
        pltpu.make_async_copy(ones_v, acc_d.at[dst_v.at[j]], edsem).wait()

      pltpu.sync_copy(e_hbm.at[pl.ds((base + j) * CHUNK, CHUNK)], e_v)
      pltpu.async_copy(e_v, acc_e.at[dst_v.at[j]], edsem, add=True)
      pltpu.async_copy(ones_v, acc_d.at[dst_v.at[j]], edsem, add=True)

    return jnp.where(mine, jnp.int32(1), pend)

  def process(j, buf, pend):
    pltpu.sync_copy(buf, acc_x.at[dst_v.at[j]], add=True)
    return emine_guarded(j, pend)

  pltpu.async_copy(x2_hbm.at[src_v.at[0]], rows0, gsem0)

  def body(i, pend):
    g = i * 2
    pltpu.async_copy(x2_hbm.at[src_v.at[g + 1]], rows1, gsem1)
    pltpu.make_async_copy(x2_hbm.at[src_v.at[g]], rows0, gsem0).wait()
    pend = process(g, rows0, pend)

    @pl.when(g + 2 < nv)
    def _():
      pltpu.async_copy(x2_hbm.at[src_v.at[g + 2]], rows0, gsem0)

    pltpu.make_async_copy(x2_hbm.at[src_v.at[g + 1]], rows1, gsem1).wait()
    pend = process(g + 1, rows1, pend)
    return pend

  pend = lax.fori_loop(0, nv // 2, body, jnp.int32(0))

  @pl.when(pend > 0)
  def _():
    pltpu.make_async_copy(e_v, acc_e.at[dst_v.at[0]], edsem).wait()
    pltpu.make_async_copy(ones_v, acc_d.at[dst_v.at[0]], edsem).wait()

  plsc.subcore_barrier()

  pltpu.sync_copy(acc_x.at[pl.ds(row0, ROWS_PER_TILE)],
                  px_hbm.at[pl.ds(row0, ROWS_PER_TILE),
                            pl.ds(cid * HALF, HALF)])
  pltpu.sync_copy(acc_e.at[pl.ds(row0, ROWS_PER_TILE)],
                  ped_hbm.at[pl.ds(row0, ROWS_PER_TILE),
                             pl.ds(cid * EDGE_DIM, EDGE_DIM)])
  pltpu.sync_copy(acc_d.at[pl.ds(row0, ROWS_PER_TILE)],
                  ped_hbm.at[pl.ds(row0, ROWS_PER_TILE),
                             pl.ds(2 * EDGE_DIM + cid * DW, DW)])


@functools.cache
def _sc_scatter():
  return pl.kernel(
      _sc_scatter_kernel,
      out_type=[
          jax.ShapeDtypeStruct((N_PAD, NODE_DIM), jnp.float32),
          jax.ShapeDtypeStruct((N_PAD, NODE_DIM), jnp.float32),
      ],
      mesh=plsc.VectorSubcoreMesh(
          core_axis_name="c", subcore_axis_name="s",
          num_cores=NC, num_subcores=NS),
      scratch_types=[
          pltpu.VMEM((K, CHUNK), jnp.int32),
          pltpu.VMEM((K, CHUNK), jnp.int32),
          pltpu.VMEM((CHUNK, HALF), jnp.float32),
          pltpu.VMEM((CHUNK, HALF), jnp.float32),
          pltpu.VMEM((CHUNK, EDGE_DIM), jnp.float32),
          pltpu.VMEM((CHUNK, DW), jnp.float32),
          pltpu.VMEM_SHARED((N_PAD, HALF), jnp.float32),
          pltpu.VMEM_SHARED((N_PAD, EDGE_DIM), jnp.float32),
          pltpu.VMEM_SHARED((N_PAD, DW), jnp.float32),
      ] + [pltpu.SemaphoreType.DMA] * 3,
      compiler_params=pltpu.CompilerParams(use_tc_tiling_on_sc=False),
  )


def _dense_body(px_ref, ped_ref, x_ref, z0_ref, wpre_ref, bpre_ref,
                wupd_ref, bupd_ref, out_ref):
  sx = px_ref[...]
  ped = ped_ref[...]
  se = ped[:, :EDGE_DIM] + ped[:, EDGE_DIM:2 * EDGE_DIM]
  deg = (ped[:, 2 * EDGE_DIM:2 * EDGE_DIM + 1]
         + ped[:, 3 * EDGE_DIM:3 * EDGE_DIM + 1])
  agg = (
      jnp.dot(sx, wpre_ref[:NODE_DIM, :], preferred_element_type=jnp.float32)
      + jnp.dot(se, wpre_ref[NODE_DIM:, :],
                preferred_element_type=jnp.float32)
      + deg * bpre_ref[0, :][None, :]
  )
  z = jnp.where(deg > 0, jnp.maximum(agg, 0.0), z0_ref[...])
  h = (
      jnp.dot(x_ref[...], wupd_ref[:NODE_DIM, :],
              preferred_element_type=jnp.float32)
      + jnp.dot(z, wupd_ref[NODE_DIM:, :], preferred_element_type=jnp.float32)
      + bupd_ref[0, :][None, :]
  )
  out_ref[...] = jnp.maximum(h, 0.0)


BLK = 1024

_dense = pl.pallas_call(
    _dense_body,
    grid=(pl.cdiv(N_PAD, BLK),),
    in_specs=[
        pl.BlockSpec((BLK, NODE_DIM), lambda i: (i, 0)),
        pl.BlockSpec((BLK, NODE_DIM), lambda i: (i, 0)),
        pl.BlockSpec((BLK, NODE_DIM), lambda i: (i, 0)),
        pl.BlockSpec((BLK, Z_DIM), lambda i: (i, 0)),
        pl.BlockSpec((Z_DIM, Z_DIM), lambda i: (0, 0)),
        pl.BlockSpec((1, Z_DIM), lambda i: (0, 0)),
        pl.BlockSpec((NODE_DIM + Z_DIM, OUT_DIM), lambda i: (0, 0)),
        pl.BlockSpec((1, OUT_DIM), lambda i: (0, 0)),
    ],
    out_specs=pl.BlockSpec((BLK, OUT_DIM), lambda i: (i, 0)),
    out_shape=jax.ShapeDtypeStruct((N_NODES, OUT_DIM), jnp.float32),
)


def kernel(x, edge_attr, edge_index, z_init, W_pre, b_pre, W_upd, b_upd):
  eidx = edge_index.astype(jnp.int32).reshape(2, NCHUNKS, CHUNK)
  x2 = x.reshape(NC * N_NODES, HALF)

  px, ped = _sc_scatter()(x2, eidx, edge_attr)

  h = _dense(px, ped, x, z_init, W_pre, b_pre.reshape(1, -1),
             W_upd, b_upd.reshape(1, -1))
  return h
